# Initial kernel scaffold; baseline (speedup 1.0000x reference)
import jax
import jax.numpy as jnp
from jax.experimental import pallas as pl


def kernel(anchor, support_set):
    raise NotImplementedError("write your pallas kernel here")


# speedup vs baseline: 427.7529x; 427.7529x over previous
import functools

import jax
import jax.numpy as jnp
from jax.experimental import pallas as pl

_NEG = -1e30

_B = 32
_C = 64
_HW = 441
_QPAD = 448
_CLASSES = 5
_PER_CLASS = 2205
_CPAD = 2304
_CHUNKS = _CPAD // 128


def _knn_body(a_ref, s_ref, o_ref):
    a = a_ref[0]
    asq = jnp.sum(a * a, axis=1, keepdims=True)
    a = a * jax.lax.rsqrt(jnp.maximum(asq, 1e-30))

    s = s_ref[...]
    ssq = jnp.sum(s * s, axis=0, keepdims=True)
    s = s * jax.lax.rsqrt(jnp.maximum(ssq, 1e-30))

    tail_iota = jax.lax.broadcasted_iota(jnp.int32, (_QPAD, 128), 1)
    tail_real = _PER_CLASS - (_CHUNKS - 1) * 128

    class_sums = []
    for c in range(_CLASSES):
        sc = s[:, c * _CPAD:(c + 1) * _CPAD]
        p = jnp.dot(a, sc, preferred_element_type=jnp.float32)

        t1 = jnp.full((_QPAD, 128), _NEG, jnp.float32)
        t2 = jnp.full((_QPAD, 128), _NEG, jnp.float32)
        t3 = jnp.full((_QPAD, 128), _NEG, jnp.float32)
        for j in range(_CHUNKS):
            cj = p[:, j * 128:(j + 1) * 128]
            if j == _CHUNKS - 1:
                cj = jnp.where(tail_iota < tail_real, cj, _NEG)
            r1 = jnp.minimum(t1, cj)
            t1 = jnp.maximum(t1, cj)
            r2 = jnp.minimum(t2, r1)
            t2 = jnp.maximum(t2, r1)
            t3 = jnp.maximum(t3, r2)

        x = jnp.concatenate([t1, t2, t3], axis=1)
        iota = jax.lax.broadcasted_iota(jnp.int32, (_QPAD, 384), 1)
        acc = jnp.zeros((_QPAD, 1), jnp.float32)
        for _ in range(3):
            m = jnp.max(x, axis=1, keepdims=True)
            acc = acc + m
            first = jnp.min(
                jnp.where(x == m, iota, jnp.int32(1 << 30)),
                axis=1, keepdims=True)
            x = jnp.where(iota == first, _NEG, x)

        class_sums.append(acc)

    o_ref[0] = jnp.sum(jnp.concatenate(class_sums, axis=1), axis=0,
                       keepdims=True)


@jax.jit
def kernel(anchor, support_set):
    a = anchor.reshape(_B, _C, _HW)
    a = jnp.transpose(a, (0, 2, 1))
    a = jnp.pad(a, ((0, 0), (0, _QPAD - _HW), (0, 0)))

    s = support_set.reshape(25, _C, _HW)
    s = jnp.transpose(s, (1, 0, 2))
    s = s.reshape(_C, _CLASSES, _PER_CLASS)
    s = jnp.pad(s, ((0, 0), (0, 0), (0, _CPAD - _PER_CLASS)))
    s = s.reshape(_C, _CLASSES * _CPAD)

    out = pl.pallas_call(
        _knn_body,
        grid=(_B,),
        in_specs=[
            pl.BlockSpec((1, _QPAD, _C), lambda b: (b, 0, 0)),
            pl.BlockSpec((_C, _CLASSES * _CPAD), lambda b: (0, 0)),
        ],
        out_specs=pl.BlockSpec((1, 1, _CLASSES), lambda b: (b, 0, 0)),
        out_shape=jax.ShapeDtypeStruct((_B, 1, _CLASSES), jnp.float32),
    )(a, s)
    return out.reshape(_B, _CLASSES)


# speedup vs baseline: 532.8184x; 1.2456x over previous
import functools

import jax
import jax.numpy as jnp
from jax.experimental import pallas as pl

_NEG = -1e30
_BIG = 1 << 30

_B = 32
_C = 64
_HW = 441
_QPAD = 448
_CLASSES = 5
_PER_CLASS = 2205
_CPAD = 2304
_CHUNKS = _CPAD // 128


def _snorm_body(s_ref, o_ref):
    s = s_ref[...]
    ssq = jnp.sum(s * s, axis=0, keepdims=True)
    o_ref[...] = s * jax.lax.rsqrt(jnp.maximum(ssq, 1e-30))


def _sorted_triple(chunks):
    while len(chunks) > 4:
        his, los = [], []
        for i in range(0, len(chunks) - 1, 2):
            his.append(jnp.maximum(chunks[i], chunks[i + 1]))
            los.append(jnp.minimum(chunks[i], chunks[i + 1]))
        if len(chunks) % 2:
            his.append(chunks[-1])
        ml = los[0]
        for l in los[1:]:
            ml = jnp.maximum(ml, l)
        chunks = his + [ml]
    a, b, c = chunks[0], chunks[1], chunks[2]
    hi_ab = jnp.maximum(a, b)
    lo_ab = jnp.minimum(a, b)
    t1 = jnp.maximum(hi_ab, c)
    m = jnp.minimum(hi_ab, c)
    t2 = jnp.maximum(lo_ab, m)
    t3 = jnp.minimum(lo_ab, m)
    for d in chunks[3:]:
        r1 = jnp.minimum(t1, d)
        t1 = jnp.maximum(t1, d)
        r2 = jnp.minimum(t2, r1)
        t2 = jnp.maximum(t2, r1)
        t3 = jnp.maximum(t3, r2)
    return t1, t2, t3


def _top3sum(t1, t2, t3, iota):
    s2 = t1 + t2
    s3 = s2 + t3
    m1 = jnp.max(t1, axis=1, keepdims=True)
    a1 = jnp.min(jnp.where(t1 == m1, iota, _BIG), axis=1, keepdims=True)
    mask1 = iota == a1
    t1b = jnp.where(mask1, _NEG, t1)
    m2 = jnp.max(t1b, axis=1, keepdims=True)
    a2 = jnp.min(jnp.where(t1b == m2, iota, _BIG), axis=1, keepdims=True)
    t1c = jnp.where(iota == a2, _NEG, t1b)
    m3 = jnp.max(t1c, axis=1, keepdims=True)
    c111 = m1 + m2 + m3
    s2_excl = jnp.where(mask1, _NEG, s2)
    s2_at = jnp.where(mask1, s2, _NEG)
    c21 = jnp.maximum(
        m1 + jnp.max(s2_excl, axis=1, keepdims=True),
        m2 + jnp.max(s2_at, axis=1, keepdims=True))
    c3 = jnp.max(s3, axis=1, keepdims=True)
    return jnp.maximum(c3, jnp.maximum(c21, c111))


def _knn_body(a_ref, s_ref, o_ref):
    a = a_ref[0]
    asq = jnp.sum(a * a, axis=1, keepdims=True)
    a = a * jax.lax.rsqrt(jnp.maximum(asq, 1e-30))
    s = s_ref[...]

    iota = jax.lax.broadcasted_iota(jnp.int32, (_QPAD, 128), 1)
    tail_real = _PER_CLASS - (_CHUNKS - 1) * 128

    class_sums = []
    for c in range(_CLASSES):
        sc = s[:, c * _CPAD:(c + 1) * _CPAD]
        p = jnp.dot(a, sc, preferred_element_type=jnp.float32)
        chunks = [p[:, j * 128:(j + 1) * 128] for j in range(_CHUNKS)]
        chunks[-1] = jnp.where(iota < tail_real, chunks[-1], _NEG)
        t1, t2, t3 = _sorted_triple(chunks)
        class_sums.append(_top3sum(t1, t2, t3, iota))

    o_ref[0] = jnp.sum(jnp.concatenate(class_sums, axis=1), axis=0,
                       keepdims=True)


@jax.jit
def kernel(anchor, support_set):
    a = anchor.reshape(_B, _C, _HW)
    a = jnp.transpose(a, (0, 2, 1))
    a = jnp.pad(a, ((0, 0), (0, _QPAD - _HW), (0, 0)))

    s = support_set.reshape(25, _C, _HW)
    s = jnp.transpose(s, (1, 0, 2))
    s = s.reshape(_C, _CLASSES, _PER_CLASS)
    s = jnp.pad(s, ((0, 0), (0, 0), (0, _CPAD - _PER_CLASS)))
    s = s.reshape(_C, _CLASSES * _CPAD)

    s = pl.pallas_call(
        _snorm_body,
        grid=(1,),
        in_specs=[pl.BlockSpec((_C, _CLASSES * _CPAD), lambda i: (0, 0))],
        out_specs=pl.BlockSpec((_C, _CLASSES * _CPAD), lambda i: (0, 0)),
        out_shape=jax.ShapeDtypeStruct((_C, _CLASSES * _CPAD), jnp.float32),
    )(s)

    out = pl.pallas_call(
        _knn_body,
        grid=(_B,),
        in_specs=[
            pl.BlockSpec((1, _QPAD, _C), lambda b: (b, 0, 0)),
            pl.BlockSpec((_C, _CLASSES * _CPAD), lambda b: (0, 0)),
        ],
        out_specs=pl.BlockSpec((1, 1, _CLASSES), lambda b: (b, 0, 0)),
        out_shape=jax.ShapeDtypeStruct((_B, 1, _CLASSES), jnp.float32),
    )(a, s)
    return out.reshape(_B, _CLASSES)
